# baseline (speedup 1.0000x reference)
import functools
from itertools import combinations

import jax
import jax.numpy as jnp
from jax import lax
from jax.experimental import pallas as pl
from jax.experimental.pallas import tpu as pltpu
from jax.experimental.pallas import tpu_sc as plsc

ROWS, COLS = 16384, 16
PAIRS = list(combinations(range(COLS), 2))
TRIPLES = list(combinations(range(COLS), 3))
OUT_COLS = COLS + len(PAIRS) + len(TRIPLES)
PAIR_COL = {c: COLS + i for i, c in enumerate(PAIRS)}
TRIPLE_COL = {c: COLS + len(PAIRS) + i for i, c in enumerate(TRIPLES)}

NC, NS, L = 2, 16, 16
NW = NC * NS
ROWS_PER_W = ROWS // NW
BLK = 128
NBLK = ROWS_PER_W // BLK
GROUPS = BLK // L
OPAD = 697


@functools.partial(
    pl.kernel,
    out_type=jax.ShapeDtypeStruct((ROWS, OUT_COLS), jnp.float32),
    mesh=plsc.VectorSubcoreMesh(core_axis_name="c", subcore_axis_name="s"),
    compiler_params=pltpu.CompilerParams(
        use_tc_tiling_on_sc=False, needs_layout_passes=False),
    scratch_types=[
        pltpu.VMEM((BLK, OPAD), jnp.float32),
    ],
)
def _drastic_sc(x_hbm, out_hbm, ov):
    wid = lax.axis_index("s") * NC + lax.axis_index("c")
    row0w = wid * ROWS_PER_W
    iota = lax.iota(jnp.int32, L)
    zero = jnp.zeros((L,), jnp.float32)

    def block(i, carry):
        row0 = row0w + i * BLK
        pltpu.sync_copy(x_hbm.at[pl.ds(row0, BLK), :], ov.at[:, pl.ds(0, COLS)])

        def group(g, gcarry):
            rowids = g * L + iota
            cols = [
                plsc.load_gather(ov, [rowids, jnp.full((L,), c, jnp.int32)])
                for c in range(COLS)
            ]
            ones = [cols[c] == 1.0 for c in range(COLS)]
            for a, b in PAIRS:
                p = jnp.where(ones[b], cols[a], jnp.where(ones[a], cols[b], zero))
                plsc.store_scatter(
                    ov, [rowids, jnp.full((L,), PAIR_COL[(a, b)], jnp.int32)], p)
                p_one = jnp.logical_and(ones[a], ones[b])
                for c in range(b + 1, COLS):
                    t = jnp.where(ones[c], p, jnp.where(p_one, cols[c], zero))
                    plsc.store_scatter(
                        ov,
                        [rowids, jnp.full((L,), TRIPLE_COL[(a, b, c)], jnp.int32)],
                        t)
            return gcarry

        lax.fori_loop(0, GROUPS, group, 0)
        pltpu.sync_copy(ov.at[:, pl.ds(0, OUT_COLS)],
                        out_hbm.at[pl.ds(row0, BLK), :])
        return carry

    lax.fori_loop(0, NBLK, block, 0)


def kernel(x):
    return _drastic_sc(x)


# speedup vs baseline: 1.2108x; 1.2108x over previous
import functools
from itertools import combinations

import jax
import jax.numpy as jnp
from jax import lax
from jax.experimental import pallas as pl
from jax.experimental.pallas import tpu as pltpu
from jax.experimental.pallas import tpu_sc as plsc

ROWS, COLS = 16384, 16
PAIRS = list(combinations(range(COLS), 2))
TRIPLES = list(combinations(range(COLS), 3))
OUT_COLS = COLS + len(PAIRS) + len(TRIPLES)
PAIR_COL = {c: COLS + i for i, c in enumerate(PAIRS)}
TRIPLE_COL = {c: COLS + len(PAIRS) + i for i, c in enumerate(TRIPLES)}

NC, NS, L = 2, 16, 16
NW = NC * NS
ROWS_PER_W = ROWS // NW
BLK = L
NBLK = ROWS_PER_W // BLK
NBUF = 3
OPAD = 697


@functools.partial(
    pl.kernel,
    out_type=jax.ShapeDtypeStruct((ROWS, OUT_COLS), jnp.float32),
    mesh=plsc.VectorSubcoreMesh(core_axis_name="c", subcore_axis_name="s"),
    compiler_params=pltpu.CompilerParams(
        use_tc_tiling_on_sc=False, needs_layout_passes=False),
    scratch_types=[
        pltpu.VMEM((BLK, OPAD), jnp.float32),
        pltpu.VMEM((BLK, OPAD), jnp.float32),
        pltpu.VMEM((BLK, OPAD), jnp.float32),
        pltpu.SemaphoreType.DMA,
        pltpu.SemaphoreType.DMA,
        pltpu.SemaphoreType.DMA,
        pltpu.SemaphoreType.DMA,
        pltpu.SemaphoreType.DMA,
        pltpu.SemaphoreType.DMA,
    ],
)
def _drastic_sc(x_hbm, out_hbm, ov0, ov1, ov2,
                isem0, isem1, isem2, osem0, osem1, osem2):
    wid = lax.axis_index("s") * NC + lax.axis_index("c")
    row0w = wid * ROWS_PER_W
    iota = lax.iota(jnp.int32, L)
    zero = jnp.zeros((L,), jnp.float32)
    ovs = (ov0, ov1, ov2)
    isems = (isem0, isem1, isem2)
    osems = (osem0, osem1, osem2)

    def start_in(k, blk_i):
        pltpu.async_copy(
            x_hbm.at[pl.ds(row0w + blk_i * BLK, BLK), :],
            ovs[k].at[:, pl.ds(0, COLS)], isems[k])

    def wait_in(k, blk_i):
        pltpu.make_async_copy(
            x_hbm.at[pl.ds(row0w + blk_i * BLK, BLK), :],
            ovs[k].at[:, pl.ds(0, COLS)], isems[k]).wait()

    def start_out(k, blk_i):
        pltpu.async_copy(
            ovs[k].at[:, pl.ds(0, OUT_COLS)],
            out_hbm.at[pl.ds(row0w + blk_i * BLK, BLK), :], osems[k])

    def wait_out(k, blk_i):
        pltpu.make_async_copy(
            ovs[k].at[:, pl.ds(0, OUT_COLS)],
            out_hbm.at[pl.ds(row0w + blk_i * BLK, BLK), :], osems[k]).wait()

    def compute(ov):
        cols = [
            plsc.load_gather(ov, [iota, jnp.full((L,), c, jnp.int32)])
            for c in range(COLS)
        ]
        ones = [cols[c] == 1.0 for c in range(COLS)]
        for a, b in PAIRS:
            p = jnp.where(ones[b], cols[a], jnp.where(ones[a], cols[b], zero))
            plsc.store_scatter(
                ov, [iota, jnp.full((L,), PAIR_COL[(a, b)], jnp.int32)], p)
            p_one = jnp.logical_and(ones[a], ones[b])
            for c in range(b + 1, COLS):
                t = jnp.where(ones[c], p, jnp.where(p_one, cols[c], zero))
                plsc.store_scatter(
                    ov, [iota, jnp.full((L,), TRIPLE_COL[(a, b, c)], jnp.int32)],
                    t)

    start_in(0, 0)

    def block(i, carry):
        k = lax.rem(i, NBUF)
        kn = lax.rem(i + 1, NBUF)

        @pl.when(i + 1 < NBLK)
        def _prefetch():
            for j in range(NBUF):
                @pl.when(kn == j)
                def _(j=j):
                    @pl.when(i >= 2)
                    def _():
                        wait_out(j, i - 2)
                    start_in(j, i + 1)

        for j in range(NBUF):
            @pl.when(k == j)
            def _(j=j):
                wait_in(j, i)
                compute(ovs[j])
                start_out(j, i)

        return carry

    lax.fori_loop(0, NBLK, block, 0)
    wait_out((NBLK - 3) % NBUF, NBLK - 3)
    wait_out((NBLK - 2) % NBUF, NBLK - 2)
    wait_out((NBLK - 1) % NBUF, NBLK - 1)


def kernel(x):
    return _drastic_sc(x)
